# baseline (speedup 1.0000x reference)
import functools

import jax
import jax.numpy as jnp
from jax import lax
from jax.experimental import pallas as pl
from jax.experimental.pallas import tpu as pltpu
from jax.experimental.pallas import tpu_sc as plsc

N_CORES = 2
N_SUBCORES = 16
N_WORKERS = N_CORES * N_SUBCORES
CHUNK = 128
GROUP = 8
ZROWS = 64
GCN_LAYERS_K = 3


@functools.partial(jax.jit, static_argnames=("n", "half_d", "n_groups", "rows_per_tile", "sub_rows"))
def _gcn_call(table0, idx_r, w_r, *, n, half_d, n_groups, rows_per_tile, sub_rows):
    n_sub = rows_per_tile // sub_rows
    mesh = plsc.VectorSubcoreMesh(core_axis_name="c", subcore_axis_name="s")

    @functools.partial(
        pl.kernel,
        mesh=mesh,
        compiler_params=pltpu.CompilerParams(use_tc_tiling_on_sc=False),
        out_type=jax.ShapeDtypeStruct((2 * n, half_d), jnp.float32),
        scratch_types=[
            pltpu.VMEM_SHARED((n, half_d), jnp.float32),
            pltpu.VMEM_SHARED((n, half_d), jnp.float32),
            pltpu.VMEM((2 * GROUP, CHUNK), jnp.int32),
            pltpu.VMEM((GROUP * CHUNK, 16), jnp.float32),
            pltpu.VMEM((CHUNK, half_d), jnp.float32),
            pltpu.VMEM((CHUNK, half_d), jnp.float32),
            pltpu.VMEM((CHUNK, half_d), jnp.float32),
            pltpu.VMEM((ZROWS, half_d), jnp.float32),
            pltpu.SemaphoreType.DMA,
            pltpu.SemaphoreType.DMA,
            pltpu.SemaphoreType.DMA,
            pltpu.SemaphoreType.DMA,
            pltpu.SemaphoreType.DMA,
            pltpu.SemaphoreType.DMA,
        ],
    )
    def gcn(table_hbm, idx_hbm, w_hbm, out_hbm,
            ping_sh, pong_sh, idx_v, w_v, rows0, rows1, rows2, zer,
            gs0, gs1, gs2, ss0, ss1, ss2):
        rows_bufs = (rows0, rows1, rows2)
        gsems = (gs0, gs1, gs2)
        ssems = (ss0, ss1, ss2)
        c = lax.axis_index("c")
        s = lax.axis_index("s")
        r0 = s * rows_per_tile
        hbm0 = c * n
        stA, stB = rows0, rows1

        @plsc.parallel_loop(0, ZROWS, 1)
        def zero_body(i):
            for j in range(half_d // 16):
                zer[i, pl.ds(j * 16, 16)] = jnp.zeros((16,), jnp.float32)

        for sub in range(n_sub):
            rows = pl.ds(r0 + sub * sub_rows, sub_rows)
            hrows = pl.ds(hbm0 + r0 + sub * sub_rows, sub_rows)
            pltpu.sync_copy(table_hbm.at[hrows], stA)
            pltpu.sync_copy(stA, ping_sh.at[rows])
            pltpu.sync_copy(stA, out_hbm.at[hrows])
        for z in range(rows_per_tile // ZROWS):
            pltpu.sync_copy(zer, pong_sh.at[pl.ds(r0 + z * ZROWS, ZROWS)])
        plsc.subcore_barrier()

        def do_layer(gather_sh, acc_sh, is_last):
            def compute(g, rv):
                @plsc.parallel_loop(0, CHUNK, 1, unroll=4)
                def edge_body(i):
                    wrow = w_v[g * CHUNK + i, :]
                    for j in range(half_d // 16):
                        sl = pl.ds(j * 16, 16)
                        rv[i, sl] = rv[i, sl] * wrow

            def group_body(grp, _):
                pltpu.sync_copy(idx_hbm.at[s, pl.ds(grp * 2 * GROUP, 2 * GROUP)], idx_v)
                pltpu.sync_copy(w_hbm.at[s, pl.ds(grp * GROUP * CHUNK, GROUP * CHUNK)], w_v)

                gh = [None] * GROUP
                sh = [None] * GROUP
                gh[0] = pltpu.async_copy(gather_sh.at[idx_v.at[0]], rows_bufs[0], gsems[0])
                for g in range(GROUP):
                    b = g % 3
                    if g >= 2:
                        sh[g - 2].wait()
                    if g + 1 < GROUP:
                        nb = (g + 1) % 3
                        gh[g + 1] = pltpu.async_copy(
                            gather_sh.at[idx_v.at[2 * (g + 1)]], rows_bufs[nb], gsems[nb])
                    gh[g].wait()
                    compute(g, rows_bufs[b])
                    sh[g] = pltpu.async_copy(
                        rows_bufs[b], acc_sh.at[idx_v.at[2 * g + 1]], ssems[b], add=True)
                sh[GROUP - 2].wait()
                sh[GROUP - 1].wait()
                return 0
            lax.fori_loop(0, n_groups, group_body, 0)
            plsc.subcore_barrier()

            for sub in range(n_sub):
                rows = pl.ds(r0 + sub * sub_rows, sub_rows)
                hrows = pl.ds(hbm0 + r0 + sub * sub_rows, sub_rows)
                pltpu.sync_copy(acc_sh.at[rows], stA)
                pltpu.sync_copy(out_hbm.at[hrows], stB)

                @plsc.parallel_loop(0, sub_rows, 1)
                def add_body(i):
                    for j in range(half_d // 16):
                        sl = pl.ds(j * 16, 16)
                        v = stB[i, sl] + stA[i, sl]
                        if is_last:
                            v = v * (1.0 / (GCN_LAYERS_K + 1))
                        stB[i, sl] = v
                pltpu.sync_copy(stB, out_hbm.at[hrows])

            if not is_last:
                for z in range(rows_per_tile // ZROWS):
                    pltpu.sync_copy(zer, gather_sh.at[pl.ds(r0 + z * ZROWS, ZROWS)])
            plsc.subcore_barrier()

        do_layer(ping_sh, pong_sh, False)
        do_layer(pong_sh, ping_sh, False)
        do_layer(ping_sh, pong_sh, True)

    return gcn(table0, idx_r, w_r)


def kernel(user_emb, item_emb, edge_index, edge_weight):
    u, d = user_emb.shape
    n = u + item_emb.shape[0]
    e = edge_weight.shape[0]
    half_d = d // 2

    sub_rows = 128
    n_pad = -(-n // (N_SUBCORES * sub_rows)) * (N_SUBCORES * sub_rows)
    rows_per_tile = n_pad // N_SUBCORES

    ept = -(-e // N_SUBCORES)
    ept = -(-ept // (GROUP * CHUNK)) * (GROUP * CHUNK)
    n_chunks = ept // CHUNK
    n_groups = n_chunks // GROUP
    e_pad = ept * N_SUBCORES

    all_emb = jnp.concatenate([user_emb, item_emb], axis=0)
    all_emb = jnp.pad(all_emb, ((0, n_pad - n), (0, 0)))
    table0 = jnp.concatenate([all_emb[:, :half_d], all_emb[:, half_d:]], axis=0)

    src = edge_index[0].astype(jnp.int32)
    dst = edge_index[1].astype(jnp.int32)
    w = edge_weight.astype(jnp.float32)
    pad = e_pad - e
    if pad:
        src = jnp.pad(src, (0, pad))
        dst = jnp.pad(dst, (0, pad))
        w = jnp.pad(w, (0, pad))

    src_r = src.reshape(N_SUBCORES, n_chunks, 1, CHUNK)
    dst_r = dst.reshape(N_SUBCORES, n_chunks, 1, CHUNK)
    idx_r = jnp.concatenate([src_r, dst_r], axis=2).reshape(N_SUBCORES, 2 * n_chunks, CHUNK)
    w_r = jnp.broadcast_to(w[:, None], (e_pad, 16)).reshape(N_SUBCORES, n_chunks * CHUNK, 16)

    out = _gcn_call(table0, idx_r, w_r, n=n_pad, half_d=half_d,
                    n_groups=n_groups, rows_per_tile=rows_per_tile,
                    sub_rows=sub_rows)
    final = jnp.concatenate([out[:n], out[n_pad:n_pad + n]], axis=1)
    return final[:u], final[u:]


# speedup vs baseline: 1.2773x; 1.2773x over previous
import functools

import jax
import jax.numpy as jnp
from jax import lax
from jax.experimental import pallas as pl
from jax.experimental.pallas import tpu as pltpu
from jax.experimental.pallas import tpu_sc as plsc

N_CORES = 2
N_SUBCORES = 16
N_WORKERS = N_CORES * N_SUBCORES
CHUNK = 128
GROUP = 8
ZROWS = 64
GCN_LAYERS_K = 3


@functools.partial(jax.jit, static_argnames=("n", "half_d", "n_groups", "rows_per_tile", "sub_rows"))
def _gcn_call(table0, idx_r, w_r, *, n, half_d, n_groups, rows_per_tile, sub_rows):
    n_sub = rows_per_tile // sub_rows
    mesh = plsc.VectorSubcoreMesh(core_axis_name="c", subcore_axis_name="s")

    @functools.partial(
        pl.kernel,
        mesh=mesh,
        compiler_params=pltpu.CompilerParams(use_tc_tiling_on_sc=False),
        out_type=jax.ShapeDtypeStruct((2 * n, half_d), jnp.float32),
        scratch_types=[
            pltpu.VMEM_SHARED((n, half_d), jnp.float32),
            pltpu.VMEM_SHARED((n, half_d), jnp.float32),
            pltpu.VMEM((2 * GROUP, CHUNK), jnp.int32),
            pltpu.VMEM((GROUP, CHUNK), jnp.float32),
            pltpu.VMEM((CHUNK, half_d), jnp.float32),
            pltpu.VMEM((CHUNK, half_d), jnp.float32),
            pltpu.VMEM((CHUNK, half_d), jnp.float32),
            pltpu.VMEM((ZROWS, half_d), jnp.float32),
            pltpu.SemaphoreType.DMA,
            pltpu.SemaphoreType.DMA,
            pltpu.SemaphoreType.DMA,
            pltpu.SemaphoreType.DMA,
            pltpu.SemaphoreType.DMA,
            pltpu.SemaphoreType.DMA,
            pltpu.SemaphoreType.DMA,
        ],
    )
    def gcn(table_hbm, idx_hbm, w_hbm, out_hbm,
            ping_sh, pong_sh, idx_v, w_v, rows0, rows1, rows2, zer,
            gs0, gs1, gs2, ss0, ss1, ss2, stsem):
        rows_bufs = (rows0, rows1, rows2)
        gsems = (gs0, gs1, gs2)
        ssems = (ss0, ss1, ss2)
        c = lax.axis_index("c")
        s = lax.axis_index("s")
        r0 = s * rows_per_tile
        hbm0 = c * n
        stA, stB = rows0, rows1

        @plsc.parallel_loop(0, ZROWS, 1)
        def zero_body(i):
            for j in range(half_d // 16):
                zer[i, pl.ds(j * 16, 16)] = jnp.zeros((16,), jnp.float32)

        for sub in range(n_sub):
            rows = pl.ds(r0 + sub * sub_rows, sub_rows)
            hrows = pl.ds(hbm0 + r0 + sub * sub_rows, sub_rows)
            pltpu.sync_copy(table_hbm.at[hrows], stA)
            pltpu.sync_copy(stA, ping_sh.at[rows])
            pltpu.sync_copy(stA, out_hbm.at[hrows])
        for z in range(rows_per_tile // ZROWS):
            pltpu.sync_copy(zer, pong_sh.at[pl.ds(r0 + z * ZROWS, ZROWS)])
        plsc.subcore_barrier()

        def do_layer(gather_sh, acc_sh, is_last):
            def compute(g, rv):
                @plsc.parallel_loop(0, CHUNK // 16, 1)
                def edge16(kk):
                    wvec = w_v[g, pl.ds(kk * 16, 16)]
                    for e in range(16):
                        i = kk * 16 + e
                        wv = wvec[e]
                        for j in range(half_d // 16):
                            sl = pl.ds(j * 16, 16)
                            rv[i, sl] = rv[i, sl] * wv

            def group_body(grp, _):
                h1 = pltpu.async_copy(idx_hbm.at[s, pl.ds(grp * 2 * GROUP, 2 * GROUP)], idx_v, stsem)
                h2 = pltpu.async_copy(w_hbm.at[s, pl.ds(grp * GROUP, GROUP)], w_v, stsem)
                h1.wait()
                h2.wait()

                gh = [None] * GROUP
                sh = [None] * GROUP
                gh[0] = pltpu.async_copy(gather_sh.at[idx_v.at[0]], rows_bufs[0], gsems[0])
                for g in range(GROUP):
                    b = g % 3
                    if g >= 2:
                        sh[g - 2].wait()
                    if g + 1 < GROUP:
                        nb = (g + 1) % 3
                        gh[g + 1] = pltpu.async_copy(
                            gather_sh.at[idx_v.at[2 * (g + 1)]], rows_bufs[nb], gsems[nb])
                    gh[g].wait()
                    compute(g, rows_bufs[b])
                    sh[g] = pltpu.async_copy(
                        rows_bufs[b], acc_sh.at[idx_v.at[2 * g + 1]], ssems[b], add=True)
                sh[GROUP - 2].wait()
                sh[GROUP - 1].wait()
                return 0
            lax.fori_loop(0, n_groups, group_body, 0)
            plsc.subcore_barrier()

            for sub in range(n_sub):
                rows = pl.ds(r0 + sub * sub_rows, sub_rows)
                hrows = pl.ds(hbm0 + r0 + sub * sub_rows, sub_rows)
                pltpu.sync_copy(acc_sh.at[rows], stA)
                pltpu.sync_copy(out_hbm.at[hrows], stB)

                @plsc.parallel_loop(0, sub_rows, 1)
                def add_body(i):
                    for j in range(half_d // 16):
                        sl = pl.ds(j * 16, 16)
                        v = stB[i, sl] + stA[i, sl]
                        if is_last:
                            v = v * (1.0 / (GCN_LAYERS_K + 1))
                        stB[i, sl] = v
                pltpu.sync_copy(stB, out_hbm.at[hrows])

            if not is_last:
                for z in range(rows_per_tile // ZROWS):
                    pltpu.sync_copy(zer, gather_sh.at[pl.ds(r0 + z * ZROWS, ZROWS)])
            plsc.subcore_barrier()

        do_layer(ping_sh, pong_sh, False)
        do_layer(pong_sh, ping_sh, False)
        do_layer(ping_sh, pong_sh, True)

    return gcn(table0, idx_r, w_r)


def kernel(user_emb, item_emb, edge_index, edge_weight):
    u, d = user_emb.shape
    n = u + item_emb.shape[0]
    e = edge_weight.shape[0]
    half_d = d // 2

    sub_rows = 128
    n_pad = -(-n // (N_SUBCORES * sub_rows)) * (N_SUBCORES * sub_rows)
    rows_per_tile = n_pad // N_SUBCORES

    ept = -(-e // N_SUBCORES)
    ept = -(-ept // (GROUP * CHUNK)) * (GROUP * CHUNK)
    n_chunks = ept // CHUNK
    n_groups = n_chunks // GROUP
    e_pad = ept * N_SUBCORES

    all_emb = jnp.concatenate([user_emb, item_emb], axis=0)
    all_emb = jnp.pad(all_emb, ((0, n_pad - n), (0, 0)))
    table0 = jnp.concatenate([all_emb[:, :half_d], all_emb[:, half_d:]], axis=0)

    src = edge_index[0].astype(jnp.int32)
    dst = edge_index[1].astype(jnp.int32)
    w = edge_weight.astype(jnp.float32)
    pad = e_pad - e
    if pad:
        src = jnp.pad(src, (0, pad))
        dst = jnp.pad(dst, (0, pad))
        w = jnp.pad(w, (0, pad))

    src_r = src.reshape(N_SUBCORES, n_chunks, 1, CHUNK)
    dst_r = dst.reshape(N_SUBCORES, n_chunks, 1, CHUNK)
    idx_r = jnp.concatenate([src_r, dst_r], axis=2).reshape(N_SUBCORES, 2 * n_chunks, CHUNK)
    w_r = w.reshape(N_SUBCORES, n_chunks, CHUNK)

    out = _gcn_call(table0, idx_r, w_r, n=n_pad, half_d=half_d,
                    n_groups=n_groups, rows_per_tile=rows_per_tile,
                    sub_rows=sub_rows)
    final = jnp.concatenate([out[:n], out[n_pad:n_pad + n]], axis=1)
    return final[:u], final[u:]


# speedup vs baseline: 1.3011x; 1.0186x over previous
import functools

import jax
import jax.numpy as jnp
from jax import lax
from jax.experimental import pallas as pl
from jax.experimental.pallas import tpu as pltpu
from jax.experimental.pallas import tpu_sc as plsc

N_CORES = 2
N_SUBCORES = 16
N_WORKERS = N_CORES * N_SUBCORES
CHUNK = 128
GROUP = 8
ZROWS = 64
GCN_LAYERS_K = 3


@functools.partial(jax.jit, static_argnames=("n", "half_d", "n_groups", "rows_per_tile", "sub_rows"))
def _gcn_call(table0, idx_r, w_r, *, n, half_d, n_groups, rows_per_tile, sub_rows):
    n_sub = rows_per_tile // sub_rows
    mesh = plsc.VectorSubcoreMesh(core_axis_name="c", subcore_axis_name="s")

    @functools.partial(
        pl.kernel,
        mesh=mesh,
        compiler_params=pltpu.CompilerParams(use_tc_tiling_on_sc=False),
        out_type=jax.ShapeDtypeStruct((2 * n, half_d), jnp.float32),
        scratch_types=[
            pltpu.VMEM_SHARED((n, half_d), jnp.float32),
            pltpu.VMEM_SHARED((n, half_d), jnp.float32),
            pltpu.VMEM((2 * GROUP, CHUNK), jnp.int32),
            pltpu.VMEM((GROUP, CHUNK), jnp.float32),
            pltpu.VMEM((CHUNK, half_d), jnp.float32),
            pltpu.VMEM((CHUNK, half_d), jnp.float32),
            pltpu.VMEM((CHUNK, half_d), jnp.float32),
            pltpu.VMEM((CHUNK, half_d), jnp.float32),
            pltpu.VMEM((ZROWS, half_d), jnp.float32),
            pltpu.SemaphoreType.DMA,
            pltpu.SemaphoreType.DMA,
            pltpu.SemaphoreType.DMA,
            pltpu.SemaphoreType.DMA,
            pltpu.SemaphoreType.DMA,
            pltpu.SemaphoreType.DMA,
            pltpu.SemaphoreType.DMA,
            pltpu.SemaphoreType.DMA,
        ],
    )
    def gcn(table_hbm, idx_hbm, w_hbm, out_hbm,
            ping_sh, pong_sh, idx_v, w_v, rows0, rows1, rows2, rows3, zer,
            gs0, gs1, gs2, ss0, ss1, ss2, stsem, wbsem):
        rows_bufs = (rows0, rows1, rows2)
        upd_bufs = (rows0, rows1, rows2, rows3)
        gsems = (gs0, gs1, gs2)
        ssems = (ss0, ss1, ss2)
        c = lax.axis_index("c")
        s = lax.axis_index("s")
        r0 = s * rows_per_tile
        hbm0 = c * n
        stA, stB = rows0, rows1

        @plsc.parallel_loop(0, ZROWS, 1)
        def zero_body(i):
            for j in range(half_d // 16):
                zer[i, pl.ds(j * 16, 16)] = jnp.zeros((16,), jnp.float32)

        for sub in range(n_sub):
            rows = pl.ds(r0 + sub * sub_rows, sub_rows)
            hrows = pl.ds(hbm0 + r0 + sub * sub_rows, sub_rows)
            pltpu.sync_copy(table_hbm.at[hrows], stA)
            pltpu.sync_copy(stA, ping_sh.at[rows])
            pltpu.sync_copy(stA, out_hbm.at[hrows])
        for z in range(rows_per_tile // ZROWS):
            pltpu.sync_copy(zer, pong_sh.at[pl.ds(r0 + z * ZROWS, ZROWS)])
        plsc.subcore_barrier()

        def do_layer(gather_sh, acc_sh, is_last):
            def compute(g, rv):
                @plsc.parallel_loop(0, CHUNK // 16, 1)
                def edge16(kk):
                    wvec = w_v[g, pl.ds(kk * 16, 16)]
                    for e in range(16):
                        i = kk * 16 + e
                        wv = wvec[e]
                        for j in range(half_d // 16):
                            sl = pl.ds(j * 16, 16)
                            rv[i, sl] = rv[i, sl] * wv

            def group_body(grp, _):
                h1 = pltpu.async_copy(idx_hbm.at[s, pl.ds(grp * 2 * GROUP, 2 * GROUP)], idx_v, stsem)
                h2 = pltpu.async_copy(w_hbm.at[s, pl.ds(grp * GROUP, GROUP)], w_v, stsem)
                h1.wait()
                h2.wait()

                gh = [None] * GROUP
                sh = [None] * GROUP
                gh[0] = pltpu.async_copy(gather_sh.at[idx_v.at[0]], rows_bufs[0], gsems[0])
                for g in range(GROUP):
                    b = g % 3
                    if g >= 2:
                        sh[g - 2].wait()
                    if g + 1 < GROUP:
                        nb = (g + 1) % 3
                        gh[g + 1] = pltpu.async_copy(
                            gather_sh.at[idx_v.at[2 * (g + 1)]], rows_bufs[nb], gsems[nb])
                    gh[g].wait()
                    compute(g, rows_bufs[b])
                    sh[g] = pltpu.async_copy(
                        rows_bufs[b], acc_sh.at[idx_v.at[2 * g + 1]], ssems[b], add=True)
                sh[GROUP - 2].wait()
                sh[GROUP - 1].wait()
                return 0
            lax.fori_loop(0, n_groups, group_body, 0)
            plsc.subcore_barrier()

            ra = [None] * n_sub
            rb = [None] * n_sub
            wb = [None] * n_sub
            for sub in range(n_sub):
                a = upd_bufs[(2 * sub) % 4]
                b = upd_bufs[(2 * sub + 1) % 4]
                rows = pl.ds(r0 + sub * sub_rows, sub_rows)
                hrows = pl.ds(hbm0 + r0 + sub * sub_rows, sub_rows)
                if sub >= 2:
                    wb[sub - 2].wait()
                ra[sub] = pltpu.async_copy(acc_sh.at[rows], a, gsems[sub % 3])
                rb[sub] = pltpu.async_copy(out_hbm.at[hrows], b, ssems[sub % 3])
                ra[sub].wait()
                rb[sub].wait()

                @plsc.parallel_loop(0, sub_rows, 1)
                def add_body(i, a=a, b=b):
                    for j in range(half_d // 16):
                        sl = pl.ds(j * 16, 16)
                        v = b[i, sl] + a[i, sl]
                        if is_last:
                            v = v * (1.0 / (GCN_LAYERS_K + 1))
                        b[i, sl] = v
                wb[sub] = pltpu.async_copy(b, out_hbm.at[hrows],
                                           stsem if sub % 2 == 0 else wbsem)
            wb[n_sub - 2].wait()
            wb[n_sub - 1].wait()

            if not is_last:
                for z in range(rows_per_tile // ZROWS):
                    pltpu.sync_copy(zer, gather_sh.at[pl.ds(r0 + z * ZROWS, ZROWS)])
            plsc.subcore_barrier()

        do_layer(ping_sh, pong_sh, False)
        do_layer(pong_sh, ping_sh, False)
        do_layer(ping_sh, pong_sh, True)

    return gcn(table0, idx_r, w_r)


def kernel(user_emb, item_emb, edge_index, edge_weight):
    u, d = user_emb.shape
    n = u + item_emb.shape[0]
    e = edge_weight.shape[0]
    half_d = d // 2

    sub_rows = 128
    n_pad = -(-n // (N_SUBCORES * sub_rows)) * (N_SUBCORES * sub_rows)
    rows_per_tile = n_pad // N_SUBCORES

    ept = -(-e // N_SUBCORES)
    ept = -(-ept // (GROUP * CHUNK)) * (GROUP * CHUNK)
    n_chunks = ept // CHUNK
    n_groups = n_chunks // GROUP
    e_pad = ept * N_SUBCORES

    all_emb = jnp.concatenate([user_emb, item_emb], axis=0)
    all_emb = jnp.pad(all_emb, ((0, n_pad - n), (0, 0)))
    table0 = jnp.concatenate([all_emb[:, :half_d], all_emb[:, half_d:]], axis=0)

    src = edge_index[0].astype(jnp.int32)
    dst = edge_index[1].astype(jnp.int32)
    w = edge_weight.astype(jnp.float32)
    pad = e_pad - e
    if pad:
        src = jnp.pad(src, (0, pad))
        dst = jnp.pad(dst, (0, pad))
        w = jnp.pad(w, (0, pad))

    src_r = src.reshape(N_SUBCORES, n_chunks, 1, CHUNK)
    dst_r = dst.reshape(N_SUBCORES, n_chunks, 1, CHUNK)
    idx_r = jnp.concatenate([src_r, dst_r], axis=2).reshape(N_SUBCORES, 2 * n_chunks, CHUNK)
    w_r = w.reshape(N_SUBCORES, n_chunks, CHUNK)

    out = _gcn_call(table0, idx_r, w_r, n=n_pad, half_d=half_d,
                    n_groups=n_groups, rows_per_tile=rows_per_tile,
                    sub_rows=sub_rows)
    final = jnp.concatenate([out[:n], out[n_pad:n_pad + n]], axis=1)
    return final[:u], final[u:]


# speedup vs baseline: 1.3046x; 1.0027x over previous
import functools

import jax
import jax.numpy as jnp
from jax import lax
from jax.experimental import pallas as pl
from jax.experimental.pallas import tpu as pltpu
from jax.experimental.pallas import tpu_sc as plsc

N_CORES = 2
N_SUBCORES = 16
N_WORKERS = N_CORES * N_SUBCORES
CHUNK = 128
GROUP = 8
ZROWS = 128
GCN_LAYERS_K = 3


@functools.partial(jax.jit, static_argnames=("n", "half_d", "n_groups", "rows_per_tile", "sub_rows"))
def _gcn_call(table0, idx_r, w_r, *, n, half_d, n_groups, rows_per_tile, sub_rows):
    n_sub = rows_per_tile // sub_rows
    mesh = plsc.VectorSubcoreMesh(core_axis_name="c", subcore_axis_name="s")

    @functools.partial(
        pl.kernel,
        mesh=mesh,
        compiler_params=pltpu.CompilerParams(use_tc_tiling_on_sc=False),
        out_type=jax.ShapeDtypeStruct((2 * n, half_d), jnp.float32),
        scratch_types=[
            pltpu.VMEM_SHARED((n, half_d), jnp.float32),
            pltpu.VMEM_SHARED((n, half_d), jnp.float32),
            pltpu.VMEM((2 * GROUP, CHUNK), jnp.int32),
            pltpu.VMEM((GROUP, CHUNK), jnp.float32),
            pltpu.VMEM((CHUNK, half_d), jnp.float32),
            pltpu.VMEM((CHUNK, half_d), jnp.float32),
            pltpu.VMEM((CHUNK, half_d), jnp.float32),
            pltpu.VMEM((CHUNK, half_d), jnp.float32),
            pltpu.VMEM((ZROWS, half_d), jnp.float32),
            pltpu.SemaphoreType.DMA,
            pltpu.SemaphoreType.DMA,
            pltpu.SemaphoreType.DMA,
            pltpu.SemaphoreType.DMA,
            pltpu.SemaphoreType.DMA,
            pltpu.SemaphoreType.DMA,
            pltpu.SemaphoreType.DMA,
            pltpu.SemaphoreType.DMA,
        ],
    )
    def gcn(table_hbm, idx_hbm, w_hbm, out_hbm,
            ping_sh, pong_sh, idx_v, w_v, rows0, rows1, rows2, rows3, zer,
            gs0, gs1, gs2, ss0, ss1, ss2, stsem, wbsem):
        rows_bufs = (rows0, rows1, rows2)
        upd_bufs = (rows0, rows1, rows2, rows3)
        gsems = (gs0, gs1, gs2)
        ssems = (ss0, ss1, ss2)
        c = lax.axis_index("c")
        s = lax.axis_index("s")
        r0 = s * rows_per_tile
        hbm0 = c * n
        stA, stB = rows0, rows1

        @plsc.parallel_loop(0, ZROWS, 1)
        def zero_body(i):
            for j in range(half_d // 16):
                zer[i, pl.ds(j * 16, 16)] = jnp.zeros((16,), jnp.float32)

        rd = [None] * n_sub
        w1 = [None] * n_sub
        w2 = [None] * n_sub
        rd[0] = pltpu.async_copy(
            table_hbm.at[pl.ds(hbm0 + r0, sub_rows)], upd_bufs[0], gsems[0])
        for sub in range(n_sub):
            a = upd_bufs[sub % 3]
            rows = pl.ds(r0 + sub * sub_rows, sub_rows)
            hrows = pl.ds(hbm0 + r0 + sub * sub_rows, sub_rows)
            if sub >= 2:
                w1[sub - 2].wait()
                w2[sub - 2].wait()
            if sub + 1 < n_sub:
                rd[sub + 1] = pltpu.async_copy(
                    table_hbm.at[pl.ds(hbm0 + r0 + (sub + 1) * sub_rows, sub_rows)],
                    upd_bufs[(sub + 1) % 3], gsems[(sub + 1) % 3])
            rd[sub].wait()
            w1[sub] = pltpu.async_copy(a, ping_sh.at[rows], ssems[sub % 3])
            w2[sub] = pltpu.async_copy(a, out_hbm.at[hrows],
                                       stsem if sub % 2 == 0 else wbsem)
        zh = []
        for z in range(rows_per_tile // ZROWS):
            zh.append(pltpu.async_copy(
                zer, pong_sh.at[pl.ds(r0 + z * ZROWS, ZROWS)], gsems[z % 2]))
        for h in w1[n_sub - 2:] + w2[n_sub - 2:] + zh:
            h.wait()
        plsc.subcore_barrier()

        def do_layer(gather_sh, acc_sh, is_last):
            def compute(g, rv):
                @plsc.parallel_loop(0, CHUNK // 16, 1)
                def edge16(kk):
                    wvec = w_v[g, pl.ds(kk * 16, 16)]
                    for e in range(16):
                        i = kk * 16 + e
                        wv = wvec[e]
                        for j in range(half_d // 16):
                            sl = pl.ds(j * 16, 16)
                            rv[i, sl] = rv[i, sl] * wv

            def group_body(grp, _):
                h1 = pltpu.async_copy(idx_hbm.at[s, pl.ds(grp * 2 * GROUP, 2 * GROUP)], idx_v, stsem)
                h2 = pltpu.async_copy(w_hbm.at[s, pl.ds(grp * GROUP, GROUP)], w_v, stsem)
                h1.wait()
                h2.wait()

                gh = [None] * GROUP
                sh = [None] * GROUP
                gh[0] = pltpu.async_copy(gather_sh.at[idx_v.at[0]], rows_bufs[0], gsems[0])
                for g in range(GROUP):
                    b = g % 3
                    if g >= 2:
                        sh[g - 2].wait()
                    if g + 1 < GROUP:
                        nb = (g + 1) % 3
                        gh[g + 1] = pltpu.async_copy(
                            gather_sh.at[idx_v.at[2 * (g + 1)]], rows_bufs[nb], gsems[nb])
                    gh[g].wait()
                    compute(g, rows_bufs[b])
                    sh[g] = pltpu.async_copy(
                        rows_bufs[b], acc_sh.at[idx_v.at[2 * g + 1]], ssems[b], add=True)
                sh[GROUP - 2].wait()
                sh[GROUP - 1].wait()
                return 0
            lax.fori_loop(0, n_groups, group_body, 0)
            plsc.subcore_barrier()

            ra = [None] * n_sub
            rb = [None] * n_sub
            wb = [None] * n_sub
            for sub in range(n_sub):
                a = upd_bufs[(2 * sub) % 4]
                b = upd_bufs[(2 * sub + 1) % 4]
                rows = pl.ds(r0 + sub * sub_rows, sub_rows)
                hrows = pl.ds(hbm0 + r0 + sub * sub_rows, sub_rows)
                if sub >= 2:
                    wb[sub - 2].wait()
                ra[sub] = pltpu.async_copy(acc_sh.at[rows], a, gsems[sub % 3])
                rb[sub] = pltpu.async_copy(out_hbm.at[hrows], b, ssems[sub % 3])
                ra[sub].wait()
                rb[sub].wait()

                @plsc.parallel_loop(0, sub_rows, 1)
                def add_body(i, a=a, b=b):
                    for j in range(half_d // 16):
                        sl = pl.ds(j * 16, 16)
                        v = b[i, sl] + a[i, sl]
                        if is_last:
                            v = v * (1.0 / (GCN_LAYERS_K + 1))
                        b[i, sl] = v
                wb[sub] = pltpu.async_copy(b, out_hbm.at[hrows],
                                           stsem if sub % 2 == 0 else wbsem)
            wb[n_sub - 2].wait()
            wb[n_sub - 1].wait()

            if not is_last:
                zh = []
                for z in range(rows_per_tile // ZROWS):
                    zh.append(pltpu.async_copy(
                        zer, gather_sh.at[pl.ds(r0 + z * ZROWS, ZROWS)], gsems[z % 2]))
                for h in zh:
                    h.wait()
            plsc.subcore_barrier()

        do_layer(ping_sh, pong_sh, False)
        do_layer(pong_sh, ping_sh, False)
        do_layer(ping_sh, pong_sh, True)

    return gcn(table0, idx_r, w_r)


def kernel(user_emb, item_emb, edge_index, edge_weight):
    u, d = user_emb.shape
    n = u + item_emb.shape[0]
    e = edge_weight.shape[0]
    half_d = d // 2

    sub_rows = 128
    n_pad = -(-n // (N_SUBCORES * sub_rows)) * (N_SUBCORES * sub_rows)
    rows_per_tile = n_pad // N_SUBCORES

    ept = -(-e // N_SUBCORES)
    ept = -(-ept // (GROUP * CHUNK)) * (GROUP * CHUNK)
    n_chunks = ept // CHUNK
    n_groups = n_chunks // GROUP
    e_pad = ept * N_SUBCORES

    all_emb = jnp.concatenate([user_emb, item_emb], axis=0)
    all_emb = jnp.pad(all_emb, ((0, n_pad - n), (0, 0)))
    table0 = jnp.concatenate([all_emb[:, :half_d], all_emb[:, half_d:]], axis=0)

    src = edge_index[0].astype(jnp.int32)
    dst = edge_index[1].astype(jnp.int32)
    w = edge_weight.astype(jnp.float32)
    pad = e_pad - e
    if pad:
        src = jnp.pad(src, (0, pad))
        dst = jnp.pad(dst, (0, pad))
        w = jnp.pad(w, (0, pad))

    src_r = src.reshape(N_SUBCORES, n_chunks, 1, CHUNK)
    dst_r = dst.reshape(N_SUBCORES, n_chunks, 1, CHUNK)
    idx_r = jnp.concatenate([src_r, dst_r], axis=2).reshape(N_SUBCORES, 2 * n_chunks, CHUNK)
    w_r = w.reshape(N_SUBCORES, n_chunks, CHUNK)

    out = _gcn_call(table0, idx_r, w_r, n=n_pad, half_d=half_d,
                    n_groups=n_groups, rows_per_tile=rows_per_tile,
                    sub_rows=sub_rows)
    final = jnp.concatenate([out[:n], out[n_pad:n_pad + n]], axis=1)
    return final[:u], final[u:]
